# baseline (device time: 162179 ns/iter reference)
import jax
import jax.numpy as jnp
from jax import lax
from jax.experimental import pallas as pl
from jax.experimental.pallas import tpu as pltpu

N_DEV = 4


def kernel(t, W):
    m_per, k = t.shape
    n = W.shape[1]
    cm = m_per // N_DEV

    def body(t_ref, w_ref, out_ref, rs_send, rs_recv,
             rs_send_sems, rs_recv_sems, ag_send_sems, ag_recv_sems):
        p = lax.axis_index("i")

        def mod(x):
            return lax.rem(x + 4 * N_DEV, N_DEV)

        left = mod(p - 1)
        right = mod(p + 1)

        barrier = pltpu.get_barrier_semaphore()
        for nbr in (left, right):
            pl.semaphore_signal(barrier, inc=1, device_id=(nbr,),
                                device_id_type=pl.DeviceIdType.MESH)
        pl.semaphore_wait(barrier, 2)

        def t_chunk_bf16(c):
            return t_ref[pl.ds(c * cm, cm), :].astype(jnp.bfloat16)

        rs_send[0, :, :] = t_chunk_bf16(mod(p - 1))
        for s in range(N_DEV - 1):
            rdma = pltpu.make_async_remote_copy(
                src_ref=rs_send.at[s],
                dst_ref=rs_recv.at[s],
                send_sem=rs_send_sems.at[s],
                recv_sem=rs_recv_sems.at[s],
                device_id=(right,),
                device_id_type=pl.DeviceIdType.MESH,
            )
            rdma.start()
            rdma.wait()
            if s < N_DEV - 2:
                rs_send[s + 1, :, :] = rs_recv[s] + t_chunk_bf16(mod(p - s - 2))

        s_p = rs_recv[N_DEV - 2] + t_chunk_bf16(p)

        w_bf = w_ref[...].astype(jnp.bfloat16)
        y = jnp.dot(s_p, w_bf, preferred_element_type=jnp.float32)
        out_ref[pl.ds(p * cm, cm), :] = y.astype(jnp.bfloat16)

        for h in range(N_DEV - 1):
            c_send = mod(p - h)
            rdma = pltpu.make_async_remote_copy(
                src_ref=out_ref.at[pl.ds(c_send * cm, cm), :],
                dst_ref=out_ref.at[pl.ds(c_send * cm, cm), :],
                send_sem=ag_send_sems.at[h],
                recv_sem=ag_recv_sems.at[h],
                device_id=(right,),
                device_id_type=pl.DeviceIdType.MESH,
            )
            rdma.start()
            rdma.wait()

    return pl.pallas_call(
        body,
        out_shape=jax.ShapeDtypeStruct((m_per, n), jnp.bfloat16),
        in_specs=[
            pl.BlockSpec(memory_space=pltpu.VMEM),
            pl.BlockSpec(memory_space=pltpu.VMEM),
        ],
        out_specs=pl.BlockSpec(memory_space=pltpu.VMEM),
        scratch_shapes=[
            pltpu.VMEM((N_DEV - 1, cm, k), jnp.bfloat16),
            pltpu.VMEM((N_DEV - 1, cm, k), jnp.bfloat16),
            pltpu.SemaphoreType.DMA((N_DEV - 1,)),
            pltpu.SemaphoreType.DMA((N_DEV - 1,)),
            pltpu.SemaphoreType.DMA((N_DEV - 1,)),
            pltpu.SemaphoreType.DMA((N_DEV - 1,)),
        ],
        compiler_params=pltpu.CompilerParams(collective_id=0),
    )(t, W)


# device time: 94973 ns/iter; 1.7076x vs baseline; 1.7076x over previous
import jax
import jax.numpy as jnp
from jax import lax
from jax.experimental import pallas as pl
from jax.experimental.pallas import tpu as pltpu

N_DEV = 4


def kernel(t, W):
    m_per, k = t.shape
    n = W.shape[1]
    cm = m_per // N_DEV
    hm = cm // 2

    def body(t_ref, w_ref, out_ref,
             rsR_send, rsR_recv, rsL_send, rsL_recv,
             rsR_send_sems, rsR_recv_sems, rsL_send_sems, rsL_recv_sems,
             agR_send_sems, agR_recv_sems, agL_send_sems, agL_recv_sems):
        p = lax.axis_index("i")

        def mod(x):
            return lax.rem(x + 4 * N_DEV, N_DEV)

        left = mod(p - 1)
        right = mod(p + 1)

        barrier = pltpu.get_barrier_semaphore()
        for nbr in (left, right):
            pl.semaphore_signal(barrier, inc=1, device_id=(nbr,),
                                device_id_type=pl.DeviceIdType.MESH)
        pl.semaphore_wait(barrier, 2)

        def half0(c):
            return t_ref[pl.ds(c * cm, hm), :].astype(jnp.bfloat16)

        def half1(c):
            return t_ref[pl.ds(c * cm + hm, hm), :].astype(jnp.bfloat16)

        rsR_send[0, :, :] = half0(mod(p - 1))
        rsL_send[0, :, :] = half1(mod(p + 1))
        for s in range(N_DEV - 1):
            rdma_r = pltpu.make_async_remote_copy(
                src_ref=rsR_send.at[s],
                dst_ref=rsR_recv.at[s],
                send_sem=rsR_send_sems.at[s],
                recv_sem=rsR_recv_sems.at[s],
                device_id=(right,),
                device_id_type=pl.DeviceIdType.MESH,
            )
            rdma_l = pltpu.make_async_remote_copy(
                src_ref=rsL_send.at[s],
                dst_ref=rsL_recv.at[s],
                send_sem=rsL_send_sems.at[s],
                recv_sem=rsL_recv_sems.at[s],
                device_id=(left,),
                device_id_type=pl.DeviceIdType.MESH,
            )
            rdma_r.start()
            rdma_l.start()
            rdma_r.wait()
            rdma_l.wait()
            if s < N_DEV - 2:
                rsR_send[s + 1, :, :] = rsR_recv[s] + half0(mod(p - s - 2))
                rsL_send[s + 1, :, :] = rsL_recv[s] + half1(mod(p + s + 2))

        s_top = rsR_recv[N_DEV - 2] + half0(p)
        s_bot = rsL_recv[N_DEV - 2] + half1(p)

        w_bf = w_ref[...].astype(jnp.bfloat16)
        y_top = jnp.dot(s_top, w_bf, preferred_element_type=jnp.float32)
        y_bot = jnp.dot(s_bot, w_bf, preferred_element_type=jnp.float32)
        out_ref[pl.ds(p * cm, hm), :] = y_top.astype(jnp.bfloat16)
        out_ref[pl.ds(p * cm + hm, hm), :] = y_bot.astype(jnp.bfloat16)

        for h in range(N_DEV - 1):
            cR = mod(p - h)
            cL = mod(p + h)
            rdma_r = pltpu.make_async_remote_copy(
                src_ref=out_ref.at[pl.ds(cR * cm, hm), :],
                dst_ref=out_ref.at[pl.ds(cR * cm, hm), :],
                send_sem=agR_send_sems.at[h],
                recv_sem=agR_recv_sems.at[h],
                device_id=(right,),
                device_id_type=pl.DeviceIdType.MESH,
            )
            rdma_l = pltpu.make_async_remote_copy(
                src_ref=out_ref.at[pl.ds(cL * cm + hm, hm), :],
                dst_ref=out_ref.at[pl.ds(cL * cm + hm, hm), :],
                send_sem=agL_send_sems.at[h],
                recv_sem=agL_recv_sems.at[h],
                device_id=(left,),
                device_id_type=pl.DeviceIdType.MESH,
            )
            rdma_r.start()
            rdma_l.start()
            rdma_r.wait()
            rdma_l.wait()

    return pl.pallas_call(
        body,
        out_shape=jax.ShapeDtypeStruct((m_per, n), jnp.bfloat16),
        in_specs=[
            pl.BlockSpec(memory_space=pltpu.VMEM),
            pl.BlockSpec(memory_space=pltpu.VMEM),
        ],
        out_specs=pl.BlockSpec(memory_space=pltpu.VMEM),
        scratch_shapes=[
            pltpu.VMEM((N_DEV - 1, hm, k), jnp.bfloat16),
            pltpu.VMEM((N_DEV - 1, hm, k), jnp.bfloat16),
            pltpu.VMEM((N_DEV - 1, hm, k), jnp.bfloat16),
            pltpu.VMEM((N_DEV - 1, hm, k), jnp.bfloat16),
            pltpu.SemaphoreType.DMA((N_DEV - 1,)),
            pltpu.SemaphoreType.DMA((N_DEV - 1,)),
            pltpu.SemaphoreType.DMA((N_DEV - 1,)),
            pltpu.SemaphoreType.DMA((N_DEV - 1,)),
            pltpu.SemaphoreType.DMA((N_DEV - 1,)),
            pltpu.SemaphoreType.DMA((N_DEV - 1,)),
            pltpu.SemaphoreType.DMA((N_DEV - 1,)),
            pltpu.SemaphoreType.DMA((N_DEV - 1,)),
        ],
        compiler_params=pltpu.CompilerParams(collective_id=0),
    )(t, W)


# device time: 83993 ns/iter; 1.9309x vs baseline; 1.1307x over previous
import jax
import jax.numpy as jnp
from jax import lax
from jax.experimental import pallas as pl
from jax.experimental.pallas import tpu as pltpu

N_DEV = 4
N_SUB = 2


def kernel(t, W):
    m_per, k = t.shape
    n = W.shape[1]
    cm = m_per // N_DEV
    sm = cm // (2 * N_SUB)

    def body(t_ref, w_ref, out_ref,
             rsR_send, rsR_recv, rsL_send, rsL_recv,
             rsR_ssem, rsR_rsem, rsL_ssem, rsL_rsem,
             agR_ssem, agR_rsem, agL_ssem, agL_rsem):
        p = lax.axis_index("i")

        def mod(x):
            return lax.rem(x + 4 * N_DEV, N_DEV)

        left = mod(p - 1)
        right = mod(p + 1)
        pending = []

        barrier = pltpu.get_barrier_semaphore()
        for nbr in (left, right):
            pl.semaphore_signal(barrier, inc=1, device_id=(nbr,),
                                device_id_type=pl.DeviceIdType.MESH)
        pl.semaphore_wait(barrier, 2)

        def t_sub(c, off):
            return t_ref[pl.ds(c * cm + off, sm), :].astype(jnp.bfloat16)

        def rs_rdma(q, s, is_right):
            return pltpu.make_async_remote_copy(
                src_ref=(rsR_send if is_right else rsL_send).at[q, s],
                dst_ref=(rsR_recv if is_right else rsL_recv).at[q, s],
                send_sem=(rsR_ssem if is_right else rsL_ssem).at[q, s],
                recv_sem=(rsR_rsem if is_right else rsL_rsem).at[q, s],
                device_id=(right if is_right else left,),
                device_id_type=pl.DeviceIdType.MESH,
            )

        def ag_rdma(q, h, is_right):
            c = mod(p - h) if is_right else mod(p + h)
            off = c * cm + (q * sm if is_right else 2 * sm + q * sm)
            return pltpu.make_async_remote_copy(
                src_ref=out_ref.at[pl.ds(off, sm), :],
                dst_ref=out_ref.at[pl.ds(off, sm), :],
                send_sem=(agR_ssem if is_right else agL_ssem).at[q, h],
                recv_sem=(agR_rsem if is_right else agL_rsem).at[q, h],
                device_id=(right if is_right else left,),
                device_id_type=pl.DeviceIdType.MESH,
            )

        for q in range(N_SUB):
            rsR_send[q, 0, :, :] = t_sub(mod(p - 1), q * sm)
            rsL_send[q, 0, :, :] = t_sub(mod(p + 1), 2 * sm + q * sm)
        for q in range(N_SUB):
            r = rs_rdma(q, 0, True)
            l = rs_rdma(q, 0, False)
            r.start()
            l.start()
            pending += [r, l]

        w_bf = w_ref[...].astype(jnp.bfloat16)

        for s in range(N_DEV - 1):
            for q in range(N_SUB):
                for is_right in (True, False):
                    rs_rdma(q, s, is_right).wait_recv()
                    recv = (rsR_recv if is_right else rsL_recv)[q, s]
                    if s < N_DEV - 2:
                        c = mod(p - s - 2) if is_right else mod(p + s + 2)
                        off = q * sm if is_right else 2 * sm + q * sm
                        snd = rsR_send if is_right else rsL_send
                        snd[q, s + 1, :, :] = recv + t_sub(c, off)
                        nxt = rs_rdma(q, s + 1, is_right)
                        nxt.start()
                        pending.append(nxt)
                    else:
                        off = q * sm if is_right else 2 * sm + q * sm
                        acc = recv + t_sub(p, off)
                        y = jnp.dot(acc, w_bf,
                                    preferred_element_type=jnp.float32)
                        out_ref[pl.ds(p * cm + off, sm), :] = (
                            y.astype(jnp.bfloat16))
                        ag = ag_rdma(q, 0, is_right)
                        ag.start()
                        pending.append(ag)

        for h in range(N_DEV - 1):
            for q in range(N_SUB):
                for is_right in (True, False):
                    ag_rdma(q, h, is_right).wait_recv()
                    if h < N_DEV - 2:
                        nxt = ag_rdma(q, h + 1, is_right)
                        nxt.start()
                        pending.append(nxt)

        for r in pending:
            r.wait_send()

    return pl.pallas_call(
        body,
        out_shape=jax.ShapeDtypeStruct((m_per, n), jnp.bfloat16),
        in_specs=[
            pl.BlockSpec(memory_space=pltpu.VMEM),
            pl.BlockSpec(memory_space=pltpu.VMEM),
        ],
        out_specs=pl.BlockSpec(memory_space=pltpu.VMEM),
        scratch_shapes=[
            pltpu.VMEM((N_SUB, N_DEV - 1, sm, k), jnp.bfloat16),
            pltpu.VMEM((N_SUB, N_DEV - 1, sm, k), jnp.bfloat16),
            pltpu.VMEM((N_SUB, N_DEV - 1, sm, k), jnp.bfloat16),
            pltpu.VMEM((N_SUB, N_DEV - 1, sm, k), jnp.bfloat16),
            pltpu.SemaphoreType.DMA((N_SUB, N_DEV - 1)),
            pltpu.SemaphoreType.DMA((N_SUB, N_DEV - 1)),
            pltpu.SemaphoreType.DMA((N_SUB, N_DEV - 1)),
            pltpu.SemaphoreType.DMA((N_SUB, N_DEV - 1)),
            pltpu.SemaphoreType.DMA((N_SUB, N_DEV - 1)),
            pltpu.SemaphoreType.DMA((N_SUB, N_DEV - 1)),
            pltpu.SemaphoreType.DMA((N_SUB, N_DEV - 1)),
            pltpu.SemaphoreType.DMA((N_SUB, N_DEV - 1)),
        ],
        compiler_params=pltpu.CompilerParams(collective_id=0),
    )(t, W)
